# baseline (speedup 1.0000x reference)
import functools

import jax
import jax.numpy as jnp
from jax import lax
from jax.experimental import pallas as pl
from jax.experimental.pallas import tpu as pltpu
from jax.experimental.pallas import tpu_sc as plsc

BATCH = 4096
SEQ = 800
D = 64
NCLS = 2
NROWS = 1000000
CHUNK = 100
CPR = SEQ // CHUNK
LANES = 16
NQ = D // LANES
NBUF = 6
UNROLL = 10

_CB = 8192
_HALF = _CB // 2
_NBLK = (NROWS + _CB - 1) // _CB
_PAD_ROWS = _NBLK * _CB


def _detile_body(in_ref, out_ref):
    tt = in_ref[...].T
    v = jnp.concatenate([tt[:_HALF], tt[_HALF:]], axis=1)
    out_ref[...] = v.reshape(_CB * D)


def _tc_detile(table_t):
    return pl.pallas_call(
        _detile_body,
        grid=(_NBLK,),
        in_specs=[pl.BlockSpec((D, _CB), lambda b: (0, b))],
        out_specs=pl.BlockSpec((_CB * D,), lambda b: (b,)),
        out_shape=jax.ShapeDtypeStruct((_PAD_ROWS * D,), jnp.float32),
    )(table_t)


def _make_sc_pool():
    info = plsc.get_sparse_core_info()
    nc, ns = info.num_cores, info.num_subcores
    nw = nc * ns
    rpw = BATCH // nw
    chunks_pw = rpw * CPR
    qchunks = chunks_pw // 4
    mesh = plsc.VectorSubcoreMesh(core_axis_name="c", subcore_axis_name="s")

    @functools.partial(
        pl.kernel,
        mesh=mesh,
        out_type=jax.ShapeDtypeStruct((BATCH, D), jnp.float32),
        scratch_types=[
            pltpu.VMEM((2, qchunks, CHUNK), jnp.int32),
        ] + [pltpu.VMEM((CHUNK, D), jnp.float32)] * NBUF
          + [pltpu.VMEM((rpw, D), jnp.float32)]
          + [pltpu.SemaphoreType.DMA] * (NBUF + 1),
        compiler_params=pltpu.CompilerParams(use_tc_tiling_on_sc=False),
    )
    def sc_pool(x_hbm, table_hbm, out_hbm, idx_v, *rest):
        gbufs = tuple(rest[:NBUF])
        pool_v = rest[NBUF]
        isem = rest[NBUF + 1]
        sems = tuple(rest[NBUF + 2:])
        wid = lax.axis_index("s") * nc + lax.axis_index("c")
        row0 = wid * rpw
        c0 = row0 * CPR

        pltpu.sync_copy(x_hbm.at[pl.ds(c0, qchunks)], idx_v.at[0])
        pltpu.async_copy(
            x_hbm.at[pl.ds(c0 + qchunks, qchunks)], idx_v.at[1], isem)

        for b in range(NBUF):
            pltpu.async_copy(table_hbm.at[idx_v.at[0, b]], gbufs[b], sems[b])

        neg_inf = jnp.full((LANES,), -jnp.inf, dtype=jnp.float32)

        def chunk_body(gb):
            def j_body(j, carry):
                a, bb = carry
                base = j * UNROLL
                for jj in range(UNROLL):
                    tgt = a if jj % 2 == 0 else bb
                    tgt = tuple(
                        jnp.maximum(
                            tgt[q], gb[base + jj, pl.ds(q * LANES, LANES)])
                        for q in range(NQ)
                    )
                    if jj % 2 == 0:
                        a = tgt
                    else:
                        bb = tgt
                return a, bb
            a, bb = lax.fori_loop(
                0, CHUNK // UNROLL, j_body,
                ((neg_inf,) * NQ, (neg_inf,) * NQ))
            return tuple(jnp.maximum(a[q], bb[q]) for q in range(NQ))

        def group_body(g, acc):
            for b in range(NBUF):
                k = g * NBUF + b
                c = lax.rem(k, CPR)
                r = lax.div(k, CPR)
                qq = lax.div(k, qchunks)
                pltpu.make_async_copy(
                    table_hbm.at[idx_v.at[lax.rem(qq, 2), lax.rem(k, qchunks)]],
                    gbufs[b], sems[b],
                ).wait()
                cmax = chunk_body(gbufs[b])
                acc = tuple(
                    jnp.where(c == 0, cmax[q], jnp.maximum(acc[q], cmax[q]))
                    for q in range(NQ)
                )

                @pl.when((lax.rem(k, qchunks) == qchunks - NBUF)
                         & (k < chunks_pw - qchunks))
                def _():
                    pltpu.make_async_copy(
                        x_hbm.at[pl.ds(c0, qchunks)],
                        idx_v.at[lax.rem(qq + 1, 2)], isem,
                    ).wait()

                @pl.when((lax.rem(k, qchunks) == 0) & (k > 0)
                         & (k < chunks_pw - qchunks))
                def _():
                    pltpu.async_copy(
                        x_hbm.at[pl.ds(c0 + (qq + 1) * qchunks, qchunks)],
                        idx_v.at[lax.rem(qq + 1, 2)], isem,
                    )

                kn = k + NBUF
                qn = lax.div(kn, qchunks)

                @pl.when(kn < chunks_pw)
                def _():
                    pltpu.async_copy(
                        table_hbm.at[
                            idx_v.at[lax.rem(qn, 2), lax.rem(kn, qchunks)]],
                        gbufs[b], sems[b],
                    )

                @pl.when(c == CPR - 1)
                def _():
                    for q in range(NQ):
                        pool_v[r, pl.ds(q * LANES, LANES)] = acc[q]
            return acc

        ntail = chunks_pw % NBUF
        acc = lax.fori_loop(
            0, chunks_pw // NBUF, group_body, (neg_inf,) * NQ)

        for t in range(ntail):
            k = chunks_pw - ntail + t
            slot = k % NBUF
            qq = k // qchunks
            pltpu.make_async_copy(
                table_hbm.at[idx_v.at[qq % 2, k % qchunks]],
                gbufs[slot], sems[slot],
            ).wait()
            cmax = chunk_body(gbufs[slot])
            acc = tuple(jnp.maximum(acc[q], cmax[q]) for q in range(NQ))
            if k % CPR == CPR - 1:
                for q in range(NQ):
                    pool_v[k // CPR, pl.ds(q * LANES, LANES)] = acc[q]

        pltpu.sync_copy(pool_v, out_hbm.at[pl.ds(row0, rpw)])

    return sc_pool


_sc_pool = _make_sc_pool()


def _fc_body(pooled_ref, w_ref, b_ref, out_ref):
    out_ref[...] = (
        jax.lax.dot_general(
            pooled_ref[...], w_ref[...],
            (((1,), (1,)), ((), ())),
            preferred_element_type=jnp.float32,
        )
        + b_ref[...]
    )


@jax.jit
def kernel(x, embed_table, fc_w, fc_b):
    xi32 = x.astype(jnp.int32)
    xp = (xi32 & ~(_CB - 1)) | ((xi32 & (_HALF - 1)) << 1) \
        | ((xi32 >> 12) & 1)
    xi = xp.reshape(BATCH * CPR, CHUNK)
    tflat = _tc_detile(embed_table.T)
    tbl = tflat.reshape(_PAD_ROWS, D)
    pooled = _sc_pool(xi, tbl)
    out = pl.pallas_call(
        _fc_body,
        out_shape=jax.ShapeDtypeStruct((BATCH, NCLS), jnp.float32),
    )(pooled, fc_w, fc_b.reshape(1, NCLS))
    return out


# speedup vs baseline: 1.0550x; 1.0550x over previous
import functools

import jax
import jax.numpy as jnp
from jax import lax
from jax.experimental import pallas as pl
from jax.experimental.pallas import tpu as pltpu
from jax.experimental.pallas import tpu_sc as plsc

BATCH = 4096
SEQ = 800
D = 64
NCLS = 2
NROWS = 1000000
CHUNK = 100
CPR = SEQ // CHUNK
LANES = 16
NQ = D // LANES
NBUF = 6
UNROLL = 10

_CB = 16384
_HALF = _CB // 2
_NBLK = (NROWS + _CB - 1) // _CB
_PAD_ROWS = _NBLK * _CB


def _detile_body(in_ref, out_ref):
    tt = in_ref[...].T
    v = jnp.concatenate([tt[:_HALF], tt[_HALF:]], axis=1)
    out_ref[...] = v.reshape(_CB * D)


def _tc_detile(table_t):
    return pl.pallas_call(
        _detile_body,
        grid=(_NBLK,),
        in_specs=[pl.BlockSpec((D, _CB), lambda b: (0, b))],
        out_specs=pl.BlockSpec((_CB * D,), lambda b: (b,)),
        out_shape=jax.ShapeDtypeStruct((_PAD_ROWS * D,), jnp.float32),
    )(table_t)


def _make_sc_pool():
    info = plsc.get_sparse_core_info()
    nc, ns = info.num_cores, info.num_subcores
    nw = nc * ns
    rpw = BATCH // nw
    chunks_pw = rpw * CPR
    qchunks = chunks_pw // 4
    mesh = plsc.VectorSubcoreMesh(core_axis_name="c", subcore_axis_name="s")

    @functools.partial(
        pl.kernel,
        mesh=mesh,
        out_type=jax.ShapeDtypeStruct((BATCH, D), jnp.float32),
        scratch_types=[
            pltpu.VMEM((2, qchunks, CHUNK), jnp.int32),
        ] + [pltpu.VMEM((CHUNK, D), jnp.float32)] * NBUF
          + [pltpu.VMEM((rpw, D), jnp.float32)]
          + [pltpu.SemaphoreType.DMA] * (NBUF + 1),
        compiler_params=pltpu.CompilerParams(use_tc_tiling_on_sc=False),
    )
    def sc_pool(x_hbm, table_hbm, out_hbm, idx_v, *rest):
        gbufs = tuple(rest[:NBUF])
        pool_v = rest[NBUF]
        isem = rest[NBUF + 1]
        sems = tuple(rest[NBUF + 2:])
        wid = lax.axis_index("s") * nc + lax.axis_index("c")
        row0 = wid * rpw
        c0 = row0 * CPR

        pltpu.sync_copy(x_hbm.at[pl.ds(c0, qchunks)], idx_v.at[0])
        pltpu.async_copy(
            x_hbm.at[pl.ds(c0 + qchunks, qchunks)], idx_v.at[1], isem)

        for b in range(NBUF):
            pltpu.async_copy(table_hbm.at[idx_v.at[0, b]], gbufs[b], sems[b])

        neg_inf = jnp.full((LANES,), -jnp.inf, dtype=jnp.float32)

        def chunk_body(gb):
            def j_body(j, carry):
                a, bb = carry
                base = j * UNROLL
                for jj in range(UNROLL):
                    tgt = a if jj % 2 == 0 else bb
                    tgt = tuple(
                        jnp.maximum(
                            tgt[q], gb[base + jj, pl.ds(q * LANES, LANES)])
                        for q in range(NQ)
                    )
                    if jj % 2 == 0:
                        a = tgt
                    else:
                        bb = tgt
                return a, bb
            a, bb = lax.fori_loop(
                0, CHUNK // UNROLL, j_body,
                ((neg_inf,) * NQ, (neg_inf,) * NQ))
            return tuple(jnp.maximum(a[q], bb[q]) for q in range(NQ))

        def group_body(g, acc):
            for b in range(NBUF):
                k = g * NBUF + b
                c = lax.rem(k, CPR)
                r = lax.div(k, CPR)
                qq = lax.div(k, qchunks)
                pltpu.make_async_copy(
                    table_hbm.at[idx_v.at[lax.rem(qq, 2), lax.rem(k, qchunks)]],
                    gbufs[b], sems[b],
                ).wait()
                cmax = chunk_body(gbufs[b])
                acc = tuple(
                    jnp.where(c == 0, cmax[q], jnp.maximum(acc[q], cmax[q]))
                    for q in range(NQ)
                )

                @pl.when((lax.rem(k, qchunks) == qchunks - NBUF)
                         & (k < chunks_pw - qchunks))
                def _():
                    pltpu.make_async_copy(
                        x_hbm.at[pl.ds(c0, qchunks)],
                        idx_v.at[lax.rem(qq + 1, 2)], isem,
                    ).wait()

                @pl.when((lax.rem(k, qchunks) == 0) & (k > 0)
                         & (k < chunks_pw - qchunks))
                def _():
                    pltpu.async_copy(
                        x_hbm.at[pl.ds(c0 + (qq + 1) * qchunks, qchunks)],
                        idx_v.at[lax.rem(qq + 1, 2)], isem,
                    )

                kn = k + NBUF
                qn = lax.div(kn, qchunks)

                @pl.when(kn < chunks_pw)
                def _():
                    pltpu.async_copy(
                        table_hbm.at[
                            idx_v.at[lax.rem(qn, 2), lax.rem(kn, qchunks)]],
                        gbufs[b], sems[b],
                    )

                @pl.when(c == CPR - 1)
                def _():
                    for q in range(NQ):
                        pool_v[r, pl.ds(q * LANES, LANES)] = acc[q]
            return acc

        ntail = chunks_pw % NBUF
        acc = lax.fori_loop(
            0, chunks_pw // NBUF, group_body, (neg_inf,) * NQ)

        for t in range(ntail):
            k = chunks_pw - ntail + t
            slot = k % NBUF
            qq = k // qchunks
            pltpu.make_async_copy(
                table_hbm.at[idx_v.at[qq % 2, k % qchunks]],
                gbufs[slot], sems[slot],
            ).wait()
            cmax = chunk_body(gbufs[slot])
            acc = tuple(jnp.maximum(acc[q], cmax[q]) for q in range(NQ))
            if k % CPR == CPR - 1:
                for q in range(NQ):
                    pool_v[k // CPR, pl.ds(q * LANES, LANES)] = acc[q]

        pltpu.sync_copy(pool_v, out_hbm.at[pl.ds(row0, rpw)])

    return sc_pool


_sc_pool = _make_sc_pool()


def _fc_body(pooled_ref, w_ref, b_ref, out_ref):
    out_ref[...] = (
        jax.lax.dot_general(
            pooled_ref[...], w_ref[...],
            (((1,), (1,)), ((), ())),
            preferred_element_type=jnp.float32,
        )
        + b_ref[...]
    )


@jax.jit
def kernel(x, embed_table, fc_w, fc_b):
    xi32 = x.astype(jnp.int32)
    xp = (xi32 & ~(_CB - 1)) | ((xi32 & (_HALF - 1)) << 1) \
        | ((xi32 >> 13) & 1)
    xi = xp.reshape(BATCH * CPR, CHUNK)
    tflat = _tc_detile(embed_table.T)
    tbl = tflat.reshape(_PAD_ROWS, D)
    pooled = _sc_pool(xi, tbl)
    out = pl.pallas_call(
        _fc_body,
        out_shape=jax.ShapeDtypeStruct((BATCH, NCLS), jnp.float32),
    )(pooled, fc_w, fc_b.reshape(1, NCLS))
    return out


# speedup vs baseline: 1.0816x; 1.0252x over previous
import functools

import jax
import jax.numpy as jnp
from jax import lax
from jax.experimental import pallas as pl
from jax.experimental.pallas import tpu as pltpu
from jax.experimental.pallas import tpu_sc as plsc

BATCH = 4096
SEQ = 800
D = 64
NCLS = 2
NROWS = 1000000
CHUNK = 100
CPR = SEQ // CHUNK
LANES = 16
NQ = D // LANES
NBUF = 6
UNROLL = 10

_CB = 32768
_HALF = _CB // 2
_NBLK = (NROWS + _CB - 1) // _CB
_PAD_ROWS = _NBLK * _CB


def _detile_body(in_ref, out_ref):
    tt = in_ref[...].T
    v = jnp.concatenate([tt[:_HALF], tt[_HALF:]], axis=1)
    out_ref[...] = v.reshape(_CB * D)


def _tc_detile(table_t):
    return pl.pallas_call(
        _detile_body,
        grid=(_NBLK,),
        in_specs=[pl.BlockSpec((D, _CB), lambda b: (0, b))],
        out_specs=pl.BlockSpec((_CB * D,), lambda b: (b,)),
        out_shape=jax.ShapeDtypeStruct((_PAD_ROWS * D,), jnp.float32),
    )(table_t)


def _make_sc_pool():
    info = plsc.get_sparse_core_info()
    nc, ns = info.num_cores, info.num_subcores
    nw = nc * ns
    rpw = BATCH // nw
    chunks_pw = rpw * CPR
    qchunks = chunks_pw // 4
    mesh = plsc.VectorSubcoreMesh(core_axis_name="c", subcore_axis_name="s")

    @functools.partial(
        pl.kernel,
        mesh=mesh,
        out_type=jax.ShapeDtypeStruct((BATCH, D), jnp.float32),
        scratch_types=[
            pltpu.VMEM((2, qchunks, CHUNK), jnp.int32),
        ] + [pltpu.VMEM((CHUNK, D), jnp.float32)] * NBUF
          + [pltpu.VMEM((rpw, D), jnp.float32)]
          + [pltpu.SemaphoreType.DMA] * (NBUF + 1),
        compiler_params=pltpu.CompilerParams(use_tc_tiling_on_sc=False),
    )
    def sc_pool(x_hbm, table_hbm, out_hbm, idx_v, *rest):
        gbufs = tuple(rest[:NBUF])
        pool_v = rest[NBUF]
        isem = rest[NBUF + 1]
        sems = tuple(rest[NBUF + 2:])
        wid = lax.axis_index("s") * nc + lax.axis_index("c")
        row0 = wid * rpw
        c0 = row0 * CPR

        pltpu.sync_copy(x_hbm.at[pl.ds(c0, qchunks)], idx_v.at[0])
        pltpu.async_copy(
            x_hbm.at[pl.ds(c0 + qchunks, qchunks)], idx_v.at[1], isem)

        for b in range(NBUF):
            pltpu.async_copy(table_hbm.at[idx_v.at[0, b]], gbufs[b], sems[b])

        neg_inf = jnp.full((LANES,), -jnp.inf, dtype=jnp.float32)

        def chunk_body(gb):
            def j_body(j, carry):
                a, bb = carry
                base = j * UNROLL
                for jj in range(UNROLL):
                    tgt = a if jj % 2 == 0 else bb
                    tgt = tuple(
                        jnp.maximum(
                            tgt[q], gb[base + jj, pl.ds(q * LANES, LANES)])
                        for q in range(NQ)
                    )
                    if jj % 2 == 0:
                        a = tgt
                    else:
                        bb = tgt
                return a, bb
            a, bb = lax.fori_loop(
                0, CHUNK // UNROLL, j_body,
                ((neg_inf,) * NQ, (neg_inf,) * NQ))
            return tuple(jnp.maximum(a[q], bb[q]) for q in range(NQ))

        def group_body(g, acc):
            for b in range(NBUF):
                k = g * NBUF + b
                c = lax.rem(k, CPR)
                r = lax.div(k, CPR)
                qq = lax.div(k, qchunks)
                pltpu.make_async_copy(
                    table_hbm.at[idx_v.at[lax.rem(qq, 2), lax.rem(k, qchunks)]],
                    gbufs[b], sems[b],
                ).wait()
                cmax = chunk_body(gbufs[b])
                acc = tuple(
                    jnp.where(c == 0, cmax[q], jnp.maximum(acc[q], cmax[q]))
                    for q in range(NQ)
                )

                @pl.when((lax.rem(k, qchunks) == qchunks - NBUF)
                         & (k < chunks_pw - qchunks))
                def _():
                    pltpu.make_async_copy(
                        x_hbm.at[pl.ds(c0, qchunks)],
                        idx_v.at[lax.rem(qq + 1, 2)], isem,
                    ).wait()

                @pl.when((lax.rem(k, qchunks) == 0) & (k > 0)
                         & (k < chunks_pw - qchunks))
                def _():
                    pltpu.async_copy(
                        x_hbm.at[pl.ds(c0 + (qq + 1) * qchunks, qchunks)],
                        idx_v.at[lax.rem(qq + 1, 2)], isem,
                    )

                kn = k + NBUF
                qn = lax.div(kn, qchunks)

                @pl.when(kn < chunks_pw)
                def _():
                    pltpu.async_copy(
                        table_hbm.at[
                            idx_v.at[lax.rem(qn, 2), lax.rem(kn, qchunks)]],
                        gbufs[b], sems[b],
                    )

                @pl.when(c == CPR - 1)
                def _():
                    for q in range(NQ):
                        pool_v[r, pl.ds(q * LANES, LANES)] = acc[q]
            return acc

        ntail = chunks_pw % NBUF
        acc = lax.fori_loop(
            0, chunks_pw // NBUF, group_body, (neg_inf,) * NQ)

        for t in range(ntail):
            k = chunks_pw - ntail + t
            slot = k % NBUF
            qq = k // qchunks
            pltpu.make_async_copy(
                table_hbm.at[idx_v.at[qq % 2, k % qchunks]],
                gbufs[slot], sems[slot],
            ).wait()
            cmax = chunk_body(gbufs[slot])
            acc = tuple(jnp.maximum(acc[q], cmax[q]) for q in range(NQ))
            if k % CPR == CPR - 1:
                for q in range(NQ):
                    pool_v[k // CPR, pl.ds(q * LANES, LANES)] = acc[q]

        pltpu.sync_copy(pool_v, out_hbm.at[pl.ds(row0, rpw)])

    return sc_pool


_sc_pool = _make_sc_pool()


def _fc_body(pooled_ref, w_ref, b_ref, out_ref):
    out_ref[...] = (
        jax.lax.dot_general(
            pooled_ref[...], w_ref[...],
            (((1,), (1,)), ((), ())),
            preferred_element_type=jnp.float32,
        )
        + b_ref[...]
    )


@jax.jit
def kernel(x, embed_table, fc_w, fc_b):
    xi32 = x.astype(jnp.int32)
    xp = (xi32 & ~(_CB - 1)) | ((xi32 & (_HALF - 1)) << 1) \
        | ((xi32 >> 14) & 1)
    xi = xp.reshape(BATCH * CPR, CHUNK)
    tflat = _tc_detile(embed_table.T)
    tbl = tflat.reshape(_PAD_ROWS, D)
    pooled = _sc_pool(xi, tbl)
    out = pl.pallas_call(
        _fc_body,
        out_shape=jax.ShapeDtypeStruct((BATCH, NCLS), jnp.float32),
    )(pooled, fc_w, fc_b.reshape(1, NCLS))
    return out
